# Initial kernel scaffold; baseline (speedup 1.0000x reference)
import jax
import jax.numpy as jnp
from jax.experimental import pallas as pl


def kernel(action, embed_weight):
    raise NotImplementedError("write your pallas kernel here")


# speedup vs baseline: 1.2400x; 1.2400x over previous
import jax
import jax.numpy as jnp
from jax.experimental import pallas as pl

_D = 512
_BLOCK = 1024


def _body(a_ref, w_ref, o_ref):
    x = a_ref[...]
    k = x.shape[1]
    m = jnp.max(x, axis=1, keepdims=True)
    ii = jax.lax.broadcasted_iota(jnp.int32, x.shape, 1)
    idx = jnp.min(jnp.where(x == m, ii, k), axis=1, keepdims=True)
    onehot = (ii == idx).astype(jnp.float32)
    o_ref[...] = jax.lax.dot_general(
        onehot, w_ref[...], (((1,), (0,)), ((), ())),
        preferred_element_type=jnp.float32,
        precision=jax.lax.Precision.HIGHEST)


def kernel(action, embed_weight):
    b, t, k = action.shape
    rows = b * t
    a2 = action.reshape(rows, k)
    grid = rows // _BLOCK
    out = pl.pallas_call(
        _body,
        grid=(grid,),
        in_specs=[pl.BlockSpec((_BLOCK, k), lambda i: (i, 0)),
                  pl.BlockSpec((k, _D), lambda i: (0, 0))],
        out_specs=pl.BlockSpec((_BLOCK, _D), lambda i: (i, 0)),
        out_shape=jax.ShapeDtypeStruct((rows, _D), jnp.float32),
    )(a2, embed_weight)
    return out.reshape(b, t, _D)


# speedup vs baseline: 1.3933x; 1.1236x over previous
import jax
import jax.numpy as jnp
from jax.experimental import pallas as pl

_D = 512
_BLOCK = 1024


def _body(a_ref, w_ref, o_ref):
    x = a_ref[...]
    k = x.shape[1]
    m = jnp.max(x, axis=1, keepdims=True)
    ii = jax.lax.broadcasted_iota(jnp.int32, x.shape, 1)
    idx = jnp.min(jnp.where(x == m, ii, k), axis=1, keepdims=True)
    onehot = (ii == idx).astype(jnp.bfloat16)
    o_ref[...] = jax.lax.dot_general(
        onehot, w_ref[...], (((1,), (0,)), ((), ())),
        preferred_element_type=jnp.float32,
        precision=jax.lax.Precision.DEFAULT)


def kernel(action, embed_weight):
    b, t, k = action.shape
    rows = b * t
    a2 = action.reshape(rows, k)
    grid = rows // _BLOCK
    out = pl.pallas_call(
        _body,
        grid=(grid,),
        in_specs=[pl.BlockSpec((_BLOCK, k), lambda i: (i, 0)),
                  pl.BlockSpec((k, _D), lambda i: (0, 0))],
        out_specs=pl.BlockSpec((_BLOCK, _D), lambda i: (i, 0)),
        out_shape=jax.ShapeDtypeStruct((rows, _D), jnp.float32),
    )(a2, embed_weight)
    return out.reshape(b, t, _D)


# speedup vs baseline: 1.9201x; 1.3781x over previous
import jax
import jax.numpy as jnp
from jax.experimental import pallas as pl

_D = 512
_BB = 128


def _body(a_ref, w_ref, o_ref):
    w = w_ref[...]
    k = a_ref.shape[2]
    t_len = a_ref.shape[1]
    for t in range(t_len):
        x = a_ref[:, t, :]
        m = jnp.max(x, axis=1, keepdims=True)
        ii = jax.lax.broadcasted_iota(jnp.int32, x.shape, 1)
        idx = jnp.min(jnp.where(x == m, ii, k), axis=1, keepdims=True)
        onehot = (ii == idx).astype(jnp.bfloat16)
        o_ref[:, t, :] = jax.lax.dot_general(
            onehot, w, (((1,), (0,)), ((), ())),
            preferred_element_type=jnp.float32,
            precision=jax.lax.Precision.DEFAULT)


def kernel(action, embed_weight):
    b, t, k = action.shape
    grid = b // _BB
    return pl.pallas_call(
        _body,
        grid=(grid,),
        in_specs=[pl.BlockSpec((_BB, t, k), lambda i: (i, 0, 0)),
                  pl.BlockSpec((k, _D), lambda i: (0, 0))],
        out_specs=pl.BlockSpec((_BB, t, _D), lambda i: (i, 0, 0)),
        out_shape=jax.ShapeDtypeStruct((b, t, _D), jnp.float32),
    )(action, embed_weight)


# speedup vs baseline: 2.3331x; 1.2151x over previous
import jax
import jax.numpy as jnp
from jax.experimental import pallas as pl

_D = 512
_BB = 256
_TT = 8


def _body(a_ref, w_ref, o_ref):
    bb, tt, k = a_ref.shape
    x = a_ref[...].reshape(bb * tt, k)
    m = jnp.max(x, axis=1, keepdims=True)
    ii = jax.lax.broadcasted_iota(jnp.int32, x.shape, 1)
    idx = jnp.min(jnp.where(x == m, ii, k), axis=1, keepdims=True)
    onehot = (ii == idx).astype(jnp.bfloat16)
    y = jax.lax.dot_general(
        onehot, w_ref[...], (((1,), (0,)), ((), ())),
        preferred_element_type=jnp.float32,
        precision=jax.lax.Precision.DEFAULT)
    o_ref[...] = y.reshape(bb, tt, _D)


def kernel(action, embed_weight):
    b, t, k = action.shape
    grid = (b // _BB, pl.cdiv(t, _TT))
    return pl.pallas_call(
        _body,
        grid=grid,
        in_specs=[pl.BlockSpec((_BB, _TT, k), lambda i, j: (i, j, 0)),
                  pl.BlockSpec((k, _D), lambda i, j: (0, 0))],
        out_specs=pl.BlockSpec((_BB, _TT, _D), lambda i, j: (i, j, 0)),
        out_shape=jax.ShapeDtypeStruct((b, t, _D), jnp.float32),
    )(action, embed_weight)


# speedup vs baseline: 2.4177x; 1.0363x over previous
import jax
import jax.numpy as jnp
from jax.experimental import pallas as pl
from jax.experimental.pallas import tpu as pltpu

_D = 512
_BB = 512
_TT = 8


def _body(a_ref, w_ref, o_ref):
    bb, tt, k = a_ref.shape
    x = a_ref[...].reshape(bb * tt, k)
    m = jnp.max(x, axis=1, keepdims=True)
    ii = jax.lax.broadcasted_iota(jnp.int32, x.shape, 1)
    idx = jnp.min(jnp.where(x == m, ii, k), axis=1, keepdims=True)
    onehot = (ii == idx).astype(jnp.bfloat16)
    y = jax.lax.dot_general(
        onehot, w_ref[...], (((1,), (0,)), ((), ())),
        preferred_element_type=jnp.float32,
        precision=jax.lax.Precision.DEFAULT)
    o_ref[...] = y.reshape(bb, tt, _D)


def kernel(action, embed_weight):
    b, t, k = action.shape
    grid = (b // _BB, pl.cdiv(t, _TT))
    return pl.pallas_call(
        _body,
        grid=grid,
        in_specs=[pl.BlockSpec((_BB, _TT, k), lambda i, j: (i, j, 0)),
                  pl.BlockSpec((k, _D), lambda i, j: (0, 0))],
        out_specs=pl.BlockSpec((_BB, _TT, _D), lambda i, j: (i, j, 0)),
        out_shape=jax.ShapeDtypeStruct((b, t, _D), jnp.float32),
        compiler_params=pltpu.CompilerParams(
            dimension_semantics=("parallel", "parallel")),
    )(action, embed_weight)


# speedup vs baseline: 2.4464x; 1.0119x over previous
import jax
import jax.numpy as jnp
from jax.experimental import pallas as pl
from jax.experimental.pallas import tpu as pltpu

_D = 512
_BB = 1024
_TT = 8


def _body(a_ref, w_ref, o_ref):
    bb, tt, k = a_ref.shape
    x = a_ref[...].reshape(bb * tt, k)
    m = jnp.max(x, axis=1, keepdims=True)
    ii = jax.lax.broadcasted_iota(jnp.int32, x.shape, 1)
    idx = jnp.min(jnp.where(x == m, ii, k), axis=1, keepdims=True)
    onehot = (ii == idx).astype(jnp.bfloat16)
    y = jax.lax.dot_general(
        onehot, w_ref[...], (((1,), (0,)), ((), ())),
        preferred_element_type=jnp.float32,
        precision=jax.lax.Precision.DEFAULT)
    o_ref[...] = y.reshape(bb, tt, _D)


def kernel(action, embed_weight):
    b, t, k = action.shape
    grid = (b // _BB, pl.cdiv(t, _TT))
    return pl.pallas_call(
        _body,
        grid=grid,
        in_specs=[pl.BlockSpec((_BB, _TT, k), lambda i, j: (i, j, 0)),
                  pl.BlockSpec((k, _D), lambda i, j: (0, 0))],
        out_specs=pl.BlockSpec((_BB, _TT, _D), lambda i, j: (i, j, 0)),
        out_shape=jax.ShapeDtypeStruct((b, t, _D), jnp.float32),
        compiler_params=pltpu.CompilerParams(
            dimension_semantics=("parallel", "parallel")),
    )(action, embed_weight)
